# Initial kernel scaffold; baseline (speedup 1.0000x reference)
import jax
import jax.numpy as jnp
from jax.experimental import pallas as pl


def kernel(x, edge_index, W1, b1, W2, b2, W3, b3):
    raise NotImplementedError("write your pallas kernel here")


# speedup vs baseline: 17.7455x; 17.7455x over previous
import jax
import jax.numpy as jnp
from jax import lax
from jax.experimental import pallas as pl
from jax.experimental.pallas import tpu as pltpu
from jax.experimental.pallas import tpu_sc as plsc

N = 10000
E = 320000
DIN = 128
NPAD = 10240
NC = 2
NS = 16
NW = NC * NS
K = 128
EPAD = 323584
EW = EPAD // NW
NCH = EW // K
NG = K // 16
SL = NPAD // NS
EPS = 1e-8
THR = 0.1

_GD = lax.GatherDimensionNumbers(
    offset_dims=(), collapsed_slice_dims=(0,), start_index_map=(0,))


def _mesh():
    return plsc.VectorSubcoreMesh(core_axis_name="c", subcore_axis_name="s")


def _perm(v, p):
    return lax.gather(v, p[:, None], _GD, (1,),
                      unique_indices=True, indices_are_sorted=False,
                      mode=lax.GatherScatterMode.PROMISE_IN_BOUNDS)


def _group_sums(accs):
    lanes = lax.iota(jnp.int32, 16)
    v = accs
    for sh, bit in ((8, 8), (4, 4), (2, 2), (1, 1)):
        pm = jnp.bitwise_xor(lanes, sh)
        mk = (lanes & bit) == 0
        nv = []
        for i in range(len(v) // 2):
            a, b = v[2 * i], v[2 * i + 1]
            ar = a + _perm(a, pm)
            br = b + _perm(b, pm)
            nv.append(jnp.where(mk, ar, _perm(br, pm)))
        v = nv
    bitrev = (((lanes & 1) << 3) | ((lanes & 2) << 1)
              | ((lanes & 4) >> 1) | ((lanes & 8) >> 3))
    return _perm(v[0], bitrev)


_LANES = lambda: lax.iota(jnp.int32, 16)


def _sc_mask_deg_body(g_h, row3_h, col3_h,
                      idx3_h, deg_h,
                      rowv2, colv2, idxv2, growA, gcolA, growB, gcolB,
                      degloc, degloc2,
                      sgrA, sgcA, sgrB, sgcB):
    c = lax.axis_index("c")
    s = lax.axis_index("s")
    w = c * NS + s
    lanes = _LANES()

    @pl.loop(0, SL // 8)
    def _z(r):
        for jj in range(8):
            degloc[r, pl.ds(jj * 16, 16)] = jnp.zeros((16,), jnp.float32)
            degloc2[r, pl.ds(jj * 16, 16)] = jnp.zeros((16,), jnp.float32)

    pltpu.sync_copy(row3_h.at[w], rowv2)
    pltpu.sync_copy(col3_h.at[w], colv2)

    def issue_gathers(k, grow, gcol, sgr, sgc):
        pltpu.async_copy(g_h.at[rowv2.at[k]], grow, sgr)
        pltpu.async_copy(g_h.at[colv2.at[k]], gcol, sgc)

    def wait_gathers(grow, gcol, sgr, sgc):
        pltpu.make_async_copy(g_h.at[rowv2.at[0]], grow, sgr).wait()
        pltpu.make_async_copy(g_h.at[colv2.at[0]], gcol, sgc).wait()

    nullvec = N + lanes + lax.rem(w, 15) * 16

    def compute_chunk(k, grow, gcol):
        @pl.loop(0, NG)
        def _grp(t):
            lo = t * 16
            rvec = rowv2[k, pl.ds(lo, 16)]
            cvec = colv2[k, pl.ds(lo, 16)]
            accs = []
            for i in range(16):
                e = lo + i
                acc = grow[e, pl.ds(0, 16)] * gcol[e, pl.ds(0, 16)]
                for jj in range(1, DIN // 16):
                    acc = acc + grow[e, pl.ds(jj * 16, 16)] * gcol[e, pl.ds(jj * 16, 16)]
                accs.append(acc)
            sims = _group_sums(accs)
            keepb = jnp.logical_or(sims >= THR, rvec == cvec)
            idxv2[k, pl.ds(lo, 16)] = jnp.where(keepb, rvec, nullvec)
            kps = jnp.where(keepb, 1.0, 0.0).astype(jnp.float32)
            for i in range(16):
                kpv = jnp.full((16,), kps[i], jnp.float32)
                cvi = cvec[i]
                dr = lax.shift_right_logical(cvi, 7)
                do = jnp.bitwise_and(lax.shift_right_logical(cvi, 4), 7) * 16
                dl = jnp.bitwise_and(cvi, 15)
                dref = degloc if i % 2 == 0 else degloc2
                dref[dr, pl.ds(do, 16)] = dref[dr, pl.ds(do, 16)] + jnp.where(
                    lanes == dl, kpv, jnp.zeros((16,), jnp.float32))

    issue_gathers(0, growA, gcolA, sgrA, sgcA)

    @pl.loop(0, NCH)
    def _chunk(k):
        even = lax.rem(k, 2) == 0

        @pl.when(even)
        def _():
            wait_gathers(growA, gcolA, sgrA, sgcA)

            @pl.when(k + 1 < NCH)
            def _():
                issue_gathers(k + 1, growB, gcolB, sgrB, sgcB)

            compute_chunk(k, growA, gcolA)

        @pl.when(jnp.logical_not(even))
        def _():
            wait_gathers(growB, gcolB, sgrB, sgcB)

            @pl.when(k + 1 < NCH)
            def _():
                issue_gathers(k + 1, growA, gcolA, sgrA, sgcA)

            compute_chunk(k, growB, gcolB)

    @pl.loop(0, SL // 8)
    def _m(r):
        for jj in range(8):
            degloc[r, pl.ds(jj * 16, 16)] = (
                degloc[r, pl.ds(jj * 16, 16)] + degloc2[r, pl.ds(jj * 16, 16)])

    pltpu.sync_copy(idxv2, idx3_h.at[w])
    pltpu.sync_copy(degloc, deg_h.at[w])


def _sc_mask_deg(g, row3, col3):
    fn = pl.kernel(
        _sc_mask_deg_body,
        out_type=(jax.ShapeDtypeStruct((NW, NCH, K), jnp.int32),
                  jax.ShapeDtypeStruct((NW, SL // 8, 128), jnp.float32)),
        mesh=_mesh(),
        scratch_types=(
            pltpu.VMEM((NCH, K), jnp.int32),
            pltpu.VMEM((NCH, K), jnp.int32),
            pltpu.VMEM((NCH, K), jnp.int32),
            pltpu.VMEM((K, DIN), jnp.float32),
            pltpu.VMEM((K, DIN), jnp.float32),
            pltpu.VMEM((K, DIN), jnp.float32),
            pltpu.VMEM((K, DIN), jnp.float32),
            pltpu.VMEM((SL // 8, 128), jnp.float32),
            pltpu.VMEM((SL // 8, 128), jnp.float32),
            pltpu.SemaphoreType.DMA,
            pltpu.SemaphoreType.DMA,
            pltpu.SemaphoreType.DMA,
            pltpu.SemaphoreType.DMA,
        ),
    )
    return fn(g, row3, col3)


def _sc_msg_body(hh_h, idx3_h, col4_h, zacc_h,
                 parts_h,
                 idxv2, colring, hbufA, hbufB, accsh,
                 sgA, sgB, ssA, ssB, scA, scB):
    c = lax.axis_index("c")
    s = lax.axis_index("s")
    w = c * NS + s

    pltpu.sync_copy(zacc_h.at[pl.ds(s * SL, SL)], accsh.at[pl.ds(s * SL, SL)])
    plsc.subcore_barrier()

    pltpu.sync_copy(idx3_h.at[w], idxv2)
    pltpu.sync_copy(col4_h.at[w, 0], colring.at[pl.ds(0, 1)])
    pltpu.sync_copy(col4_h.at[w, 1], colring.at[pl.ds(1, 1)])

    def issue_gather(k, hbuf, sg):
        pltpu.async_copy(hh_h.at[idxv2.at[k]], hbuf, sg)

    def wait_gather(hbuf, sg):
        pltpu.make_async_copy(hh_h.at[idxv2.at[0]], hbuf, sg).wait()

    def issue_scat(k, hbuf, ss):
        pltpu.async_copy(hbuf, accsh.at[colring.at[lax.rem(k, 4)]], ss, add=True)

    def drain_scat(k, hbuf, ss):
        pltpu.make_async_copy(hbuf, accsh.at[colring.at[lax.rem(k, 4)]], ss).wait()

    def issue_col(k, sc):
        pltpu.async_copy(col4_h.at[w, k], colring.at[pl.ds(lax.rem(k, 4), 1)], sc)

    def wait_col(k, sc):
        pltpu.make_async_copy(
            col4_h.at[w, k], colring.at[pl.ds(lax.rem(k, 4), 1)], sc).wait()

    issue_gather(0, hbufA, sgA)

    @pl.loop(0, NCH)
    def _chunk(k):
        even = lax.rem(k, 2) == 0

        @pl.when(even)
        def _():
            wait_gather(hbufA, sgA)

            @pl.when(k >= 1)
            def _():
                drain_scat(k - 1, hbufB, ssB)

            @pl.when(k + 1 < NCH)
            def _():
                issue_gather(k + 1, hbufB, sgB)

            @pl.when(k + 2 < NCH)
            def _():
                issue_col(k + 2, scA)

            @pl.when(k >= 2)
            def _():
                wait_col(k, scA)

            issue_scat(k, hbufA, ssA)

        @pl.when(jnp.logical_not(even))
        def _():
            wait_gather(hbufB, sgB)
            drain_scat(k - 1, hbufA, ssA)

            @pl.when(k + 1 < NCH)
            def _():
                issue_gather(k + 1, hbufA, sgA)

            @pl.when(k + 2 < NCH)
            def _():
                issue_col(k + 2, scB)

            @pl.when(k >= 2)
            def _():
                wait_col(k, scB)

            issue_scat(k, hbufB, ssB)

    drain_scat(NCH - 1, hbufA, ssA)
    plsc.subcore_barrier()
    pltpu.sync_copy(accsh.at[pl.ds(s * SL, SL)], parts_h.at[c, pl.ds(s * SL, SL)])


def _sc_msg(hh, idx3, col4, zacc):
    fn = pl.kernel(
        _sc_msg_body,
        out_type=jax.ShapeDtypeStruct((NC, NPAD, 128), jnp.float32),
        mesh=_mesh(),
        scratch_types=(
            pltpu.VMEM((NCH, K), jnp.int32),
            pltpu.VMEM((4, K), jnp.int32),
            pltpu.VMEM((K, 128), jnp.float32),
            pltpu.VMEM((K, 128), jnp.float32),
            pltpu.VMEM_SHARED((NPAD, 128), jnp.float32),
            pltpu.SemaphoreType.DMA,
            pltpu.SemaphoreType.DMA,
            pltpu.SemaphoreType.DMA,
            pltpu.SemaphoreType.DMA,
            pltpu.SemaphoreType.DMA,
            pltpu.SemaphoreType.DMA,
        ),
    )
    return fn(hh, idx3, col4, zacc)


BR = 1024


def _rows_spec(d):
    return pl.BlockSpec((BR, d), lambda i: (i, 0))


def _full_spec(shape):
    return pl.BlockSpec(shape, lambda i: tuple(0 for _ in shape))


def _tc_prep_body(x_ref, W_ref, g_ref, h_ref):
    xb = x_ref[...]
    ss = jnp.sum(xb * xb, axis=1, keepdims=True)
    inv = 1.0 / jnp.maximum(jnp.sqrt(ss), EPS)
    g_ref[...] = xb * inv
    h_ref[...] = jnp.dot(xb, W_ref[...], preferred_element_type=jnp.float32)


def _tc_prep(x, W):
    dh = W.shape[1]
    return pl.pallas_call(
        _tc_prep_body,
        grid=(NPAD // BR,),
        in_specs=[_rows_spec(DIN), _full_spec((DIN, dh))],
        out_specs=[_rows_spec(DIN), _rows_spec(dh)],
        out_shape=[jax.ShapeDtypeStruct((NPAD, DIN), jnp.float32),
                   jax.ShapeDtypeStruct((NPAD, dh), jnp.float32)],
    )(x, W)


def _tc_mid_body(degT_ref, h_ref, dinv_ref, hh_ref):
    dt = degT_ref[...]
    deg = jnp.sum(dt, axis=1, keepdims=True) + 1.0
    dinv = lax.rsqrt(deg)
    dinv_ref[...] = dinv
    hh_ref[...] = h_ref[...] * dinv


def _tc_mid(degT, h):
    return pl.pallas_call(
        _tc_mid_body,
        grid=(NPAD // BR,),
        in_specs=[_rows_spec(NW), _rows_spec(128)],
        out_specs=[_rows_spec(1), _rows_spec(128)],
        out_shape=[jax.ShapeDtypeStruct((NPAD, 1), jnp.float32),
                   jax.ShapeDtypeStruct((NPAD, 128), jnp.float32)],
    )(degT, h)


def _tc_step_body(p0_ref, p1_ref, hh_ref, dinv_ref, b_ref, W_ref, mask_ref,
                  g_ref, h_ref):
    f = dinv_ref[...] * (p0_ref[...] + p1_ref[...] + hh_ref[...]) + b_ref[...]
    f = jnp.maximum(f, 0.0) * mask_ref[...]
    ss = jnp.sum(f * f, axis=1, keepdims=True)
    inv = 1.0 / jnp.maximum(jnp.sqrt(ss), EPS)
    g_ref[...] = f * inv
    h_ref[...] = jnp.dot(f, W_ref[...], preferred_element_type=jnp.float32)


def _tc_step(p0, p1, hh, dinv, b, W, mask):
    dh = hh.shape[1]
    dh2 = W.shape[1]
    return pl.pallas_call(
        _tc_step_body,
        grid=(NPAD // BR,),
        in_specs=[_rows_spec(dh), _rows_spec(dh), _rows_spec(dh), _rows_spec(1),
                  _full_spec((dh,)), _full_spec((dh, dh2)), _rows_spec(1)],
        out_specs=[_rows_spec(dh), _rows_spec(dh2)],
        out_shape=[jax.ShapeDtypeStruct((NPAD, dh), jnp.float32),
                   jax.ShapeDtypeStruct((NPAD, dh2), jnp.float32)],
    )(p0, p1, hh, dinv, b, W, mask)


def _tc_final_body(p0_ref, p1_ref, hh_ref, dinv_ref, b_ref, out_ref):
    o = dinv_ref[...] * (p0_ref[...] + p1_ref[...] + hh_ref[...]) + b_ref[...]
    m = jnp.max(o, axis=1, keepdims=True)
    lse = jnp.log(jnp.sum(jnp.exp(o - m), axis=1, keepdims=True)) + m
    out_ref[...] = o - lse


def _tc_final(p0, p1, hh, dinv, b):
    dh = hh.shape[1]
    return pl.pallas_call(
        _tc_final_body,
        grid=(NPAD // BR,),
        in_specs=[_rows_spec(dh), _rows_spec(dh), _rows_spec(dh), _rows_spec(1),
                  _full_spec((dh,))],
        out_specs=_rows_spec(dh),
        out_shape=jax.ShapeDtypeStruct((NPAD, dh), jnp.float32),
    )(p0, p1, hh, dinv, b)


def _layer(g, h, row3, col3, col4, zacc):
    idx3, degp = _sc_mask_deg(g, row3, col3)
    degT = degp.reshape(NW, NPAD).T
    dinv, hhp = _tc_mid(degT, h)
    partsfull = _sc_msg(hhp, idx3, col4, zacc)
    return partsfull[0], partsfull[1], hhp, dinv


def kernel(x, edge_index, W1, b1, W2, b2, W3, b3):
    assert x.shape == (N, DIN) and edge_index.shape == (2, E)
    padidx = (N + (jnp.arange(EPAD - E) % 240)).astype(edge_index.dtype)
    rowp = jnp.concatenate([edge_index[0], padidx])
    colp = jnp.concatenate([edge_index[1], padidx])
    row3 = rowp.reshape(NW, NCH, K)
    col3 = colp.reshape(NW, NCH, K)
    col4 = colp.reshape(NW, NCH, 1, K)
    xp = jnp.concatenate([x, jnp.zeros((NPAD - N, DIN), jnp.float32)], axis=0)
    mask = (jnp.arange(NPAD, dtype=jnp.int32) < N).astype(jnp.float32)[:, None]
    zacc = jnp.zeros((NPAD, 128), jnp.float32)

    dout = W3.shape[1]
    W3p = jnp.concatenate([W3, jnp.zeros((DIN, 128 - dout), jnp.float32)], axis=1)
    Ws = jnp.stack([W2, W3p, jnp.zeros((128, 128), jnp.float32)])
    bs = jnp.stack([b1, b2, jnp.zeros((128,), jnp.float32)])

    g, h = _tc_prep(xp, W1)

    niters = lax.optimization_barrier(jnp.int32(3))

    def cond(st):
        return st[0] < niters

    def body(st):
        i, g, h, _, _, _, _ = st
        p0, p1, hhp, dinv = _layer(g, h, row3, col3, col4, zacc)
        W = lax.dynamic_index_in_dim(Ws, i, keepdims=False)
        b = lax.dynamic_index_in_dim(bs, i, keepdims=False)
        g2, h2 = _tc_step(p0, p1, hhp, dinv, b, W, mask)
        return (i + 1, g2, h2, p0, p1, hhp, dinv)

    zP = jnp.zeros((NPAD, 128), jnp.float32)
    zD = jnp.zeros((NPAD, 1), jnp.float32)
    st0 = (jnp.int32(0), g, h, zP, zP, zP, zD)
    _, _, _, p0, p1, hh, dinv = lax.while_loop(cond, body, st0)
    out = _tc_final(p0[:, :dout], p1[:, :dout], hh[:, :dout], dinv, b3)
    return out[:N]
